# Initial kernel scaffold; baseline (speedup 1.0000x reference)
import functools

import jax
import jax.numpy as jnp
from jax import lax
from jax.experimental import pallas as pl
from jax.experimental.pallas import tpu as pltpu
from jax.experimental.pallas import tpu_sc as plsc

N = 10000
NPAD = 10240
E = 320000
EPAD = 331776
D = 128
HID = 128
HEADS = 8
EMB = 128
F = 144
NTILES = 32
CHUNKS = 81
CK = 128
ROWS_PER_TILE = NPAD // 16


def _leaky(u):
    return jnp.maximum(u, 0.2 * u)


def _l1_prep_body(x_ref, w1_ref, asbd_ref, adbd_ref, haug_ref, as_ref, ad_ref):
    h = jnp.dot(x_ref[...], w1_ref[...], preferred_element_type=jnp.float32)
    as_ref[...] = jnp.dot(h, asbd_ref[...], preferred_element_type=jnp.float32)
    ad_ref[...] = jnp.dot(h, adbd_ref[...], preferred_element_type=jnp.float32)
    pad = jnp.concatenate(
        [jnp.ones((128, 1), jnp.float32), jnp.zeros((128, 15), jnp.float32)], axis=1)
    for hd in range(HEADS):
        haug_ref[hd, :, 0:128] = h[:, hd * 128:(hd + 1) * 128]
        haug_ref[hd, :, 128:144] = pad


def _l1_prep(xpad, W1, asbd, adbd):
    return pl.pallas_call(
        _l1_prep_body,
        grid=(NPAD // 128,),
        in_specs=[
            pl.BlockSpec((128, D), lambda i: (i, 0)),
            pl.BlockSpec((D, HEADS * HID), lambda i: (0, 0)),
            pl.BlockSpec((HEADS * HID, HEADS), lambda i: (0, 0)),
            pl.BlockSpec((HEADS * HID, HEADS), lambda i: (0, 0)),
        ],
        out_specs=[
            pl.BlockSpec((HEADS, 128, F), lambda i: (0, i, 0)),
            pl.BlockSpec((128, HEADS), lambda i: (i, 0)),
            pl.BlockSpec((128, HEADS), lambda i: (i, 0)),
        ],
        out_shape=[
            jax.ShapeDtypeStruct((HEADS, NPAD, F), jnp.float32),
            jax.ShapeDtypeStruct((NPAD, HEADS), jnp.float32),
            jax.ShapeDtypeStruct((NPAD, HEADS), jnp.float32),
        ],
    )(xpad, W1, asbd, adbd)


def _shift_body(as_ref, ad_ref, cc_ref):
    nh = as_ref.shape[1]
    ridx = lax.broadcasted_iota(jnp.int32, (NPAD, nh), 0)
    masked = jnp.where(ridx < N, as_ref[...], -3e38)
    amax = jnp.max(masked, axis=0, keepdims=True)
    cc_ref[...] = _leaky(ad_ref[...] + amax)


def _shift_table(as2d, ad2d):
    nh = as2d.shape[1]
    return pl.pallas_call(
        _shift_body,
        out_shape=jax.ShapeDtypeStruct((NPAD, nh), jnp.float32),
    )(as2d, ad2d)


def _sc_body(nheads, htab, ast, adt, cct, srca, dsta, out_ref,
             src_v, dst_v, gidx_v, w_v, rows_v, as_v, ad_v, cc_v, zbuf_v,
             out_sp, sem):
    c = lax.axis_index("c")
    s = lax.axis_index("s")
    wid = c * 16 + s

    pltpu.sync_copy(srca.at[wid], src_v)
    pltpu.sync_copy(dsta.at[wid], dst_v)

    zero16 = jnp.zeros((16,), jnp.float32)

    def zb(i, _):
        for j in range(F // 16):
            zbuf_v[i, pl.ds(j * 16, 16)] = zero16
        return 0
    lax.fori_loop(0, 64, zb, 0)

    def head_loop(hd, _):
        pltpu.sync_copy(ast.at[hd], as_v)
        pltpu.sync_copy(adt.at[hd], ad_v)
        pltpu.sync_copy(cct.at[hd], cc_v)

        def zo(i, _):
            pltpu.sync_copy(zbuf_v, out_sp.at[pl.ds(s * ROWS_PER_TILE + i * 64, 64)])
            return 0
        lax.fori_loop(0, ROWS_PER_TILE // 64, zo, 0)
        plsc.subcore_barrier()

        base = hd * NPAD

        def chunk_loop(ch, _):
            for j in range(CK // 16):
                sl = pl.ds(j * 16, 16)
                sv = src_v[ch, sl]
                dv = dst_v[ch, sl]
                asg = plsc.load_gather(as_v, [sv])
                adg = plsc.load_gather(ad_v, [dv])
                ccg = plsc.load_gather(cc_v, [dv])
                u = asg + adg
                w_v[sl] = jnp.exp(_leaky(u) - ccg)
                gidx_v[sl] = sv + base
            pltpu.async_copy(htab.at[gidx_v], rows_v, sem).wait()

            def scale(k, _):
                wk = w_v[k]
                for j in range(F // 16):
                    fsl = pl.ds(j * 16, 16)
                    rows_v[k, fsl] = rows_v[k, fsl] * wk
                return 0
            lax.fori_loop(0, CK, scale, 0)
            pltpu.sync_copy(rows_v, out_sp.at[dst_v.at[ch]], add=True)
            return 0
        lax.fori_loop(0, CHUNKS, chunk_loop, 0)
        plsc.subcore_barrier()

        def wo(i, _):
            r = s * ROWS_PER_TILE + i * 64
            pltpu.sync_copy(out_sp.at[pl.ds(r, 64)],
                            out_ref.at[c, hd, pl.ds(r, 64)])
            return 0
        lax.fori_loop(0, ROWS_PER_TILE // 64, wo, 0)
        plsc.subcore_barrier()
        return 0
    lax.fori_loop(0, nheads, head_loop, 0)


def _sc_aggregate(nheads, htab, ast, adt, cct, srca, dsta):
    mesh = plsc.VectorSubcoreMesh(core_axis_name="c", subcore_axis_name="s")
    kfn = pl.kernel(
        functools.partial(_sc_body, nheads),
        out_type=jax.ShapeDtypeStruct((2, nheads, NPAD, F), jnp.float32),
        mesh=mesh,
        scratch_types=[
            pltpu.VMEM((CHUNKS, CK), jnp.int32),
            pltpu.VMEM((CHUNKS, CK), jnp.int32),
            pltpu.VMEM((CK,), jnp.int32),
            pltpu.VMEM((CK,), jnp.float32),
            pltpu.VMEM((CK, F), jnp.float32),
            pltpu.VMEM((NPAD,), jnp.float32),
            pltpu.VMEM((NPAD,), jnp.float32),
            pltpu.VMEM((NPAD,), jnp.float32),
            pltpu.VMEM((64, F), jnp.float32),
            pltpu.VMEM_SHARED((NPAD, F), jnp.float32),
            pltpu.SemaphoreType.DMA,
        ],
    )
    return kfn(htab, ast, adt, cct, srca, dsta)


def _l2_prep_body(p_ref, b1_ref, w2_ref, as2v_ref, ad2v_ref,
                  haug_ref, as2_ref, ad2_ref):
    h2 = jnp.zeros((128, EMB), jnp.float32)
    for hd in range(HEADS):
        num = p_ref[0, hd, :, 0:128] + p_ref[1, hd, :, 0:128]
        den = p_ref[0, hd, :, 128:129] + p_ref[1, hd, :, 128:129]
        g = num / (den + 1e-16) + b1_ref[hd:hd + 1, :]
        e = jnp.where(g > 0, g, jnp.exp(g) - 1.0)
        h2 = h2 + jnp.dot(e, w2_ref[hd * 128:(hd + 1) * 128, :],
                          preferred_element_type=jnp.float32)
    pad = jnp.concatenate(
        [jnp.ones((128, 1), jnp.float32), jnp.zeros((128, 15), jnp.float32)], axis=1)
    haug_ref[:, 0:128] = h2
    haug_ref[:, 128:144] = pad
    as2_ref[0, 0, :] = jnp.sum(h2 * as2v_ref[0:1, :], axis=1)
    ad2_ref[0, 0, :] = jnp.sum(h2 * ad2v_ref[0:1, :], axis=1)


def _l2_prep(part1, b1_2d, W2, a_src2, a_dst2):
    return pl.pallas_call(
        _l2_prep_body,
        grid=(NPAD // 128,),
        in_specs=[
            pl.BlockSpec((2, HEADS, 128, F), lambda i: (0, 0, i, 0)),
            pl.BlockSpec((HEADS, HID), lambda i: (0, 0)),
            pl.BlockSpec((HEADS * HID, EMB), lambda i: (0, 0)),
            pl.BlockSpec((1, EMB), lambda i: (0, 0)),
            pl.BlockSpec((1, EMB), lambda i: (0, 0)),
        ],
        out_specs=[
            pl.BlockSpec((128, F), lambda i: (i, 0)),
            pl.BlockSpec((1, 1, 128), lambda i: (i, 0, 0)),
            pl.BlockSpec((1, 1, 128), lambda i: (i, 0, 0)),
        ],
        out_shape=[
            jax.ShapeDtypeStruct((NPAD, F), jnp.float32),
            jax.ShapeDtypeStruct((NPAD // 128, 1, 128), jnp.float32),
            jax.ShapeDtypeStruct((NPAD // 128, 1, 128), jnp.float32),
        ],
    )(part1, b1_2d, W2, a_src2, a_dst2)


def _shift2_body(as_ref, ad_ref, cc_ref):
    ridx = (lax.broadcasted_iota(jnp.int32, (NPAD // 128, 1, 128), 0) * 128
            + lax.broadcasted_iota(jnp.int32, (NPAD // 128, 1, 128), 2))
    masked = jnp.where(ridx < N, as_ref[...], -3e38)
    amax = jnp.max(masked)
    cc_ref[...] = _leaky(ad_ref[...] + amax)


def _shift2_table(as3d, ad3d):
    return pl.pallas_call(
        _shift2_body,
        out_shape=jax.ShapeDtypeStruct((NPAD // 128, 1, 128), jnp.float32),
    )(as3d, ad3d)


def _final_body(p_ref, b2_ref, out_ref):
    num = p_ref[0, :, 0:128] + p_ref[1, :, 0:128]
    den = p_ref[0, :, 128:129] + p_ref[1, :, 128:129]
    z = num / (den + 1e-16) + b2_ref[0:1, :]
    m = jnp.max(z, axis=1, keepdims=True)
    zs = z - m
    out_ref[...] = zs - jnp.log(jnp.sum(jnp.exp(zs), axis=1, keepdims=True))


def _final(part2, b2_2d):
    return pl.pallas_call(
        _final_body,
        grid=(79,),
        in_specs=[
            pl.BlockSpec((2, 128, F), lambda i: (0, i, 0)),
            pl.BlockSpec((1, EMB), lambda i: (0, 0)),
        ],
        out_specs=pl.BlockSpec((128, EMB), lambda i: (i, 0)),
        out_shape=jax.ShapeDtypeStruct((N, EMB), jnp.float32),
    )(part2, b2_2d)


def kernel(x, edge_index, W1, a_src1, a_dst1, b1, W2, a_src2, a_dst2, b2):
    ei = edge_index.astype(jnp.int32)
    loop = jnp.arange(N, dtype=jnp.int32)
    padn = EPAD - (E + N)
    src = jnp.concatenate([ei[0], loop, jnp.zeros((padn,), jnp.int32)])
    dst = jnp.concatenate([ei[1], loop, jnp.full((padn,), N + 10, jnp.int32)])
    srca = src.reshape(NTILES, CHUNKS, CK)
    dsta = dst.reshape(NTILES, CHUNKS, CK)

    xpad = jnp.pad(x, ((0, NPAD - N), (0, 0)))

    eye = jnp.eye(HEADS, dtype=jnp.float32)
    asbd = (eye[:, None, :] * a_src1[:, :, None]).reshape(HEADS * HID, HEADS)
    adbd = (eye[:, None, :] * a_dst1[:, :, None]).reshape(HEADS * HID, HEADS)

    haug1, as1, ad1 = _l1_prep(xpad, W1, asbd, adbd)
    cc1 = _shift_table(as1, ad1)

    htab1 = haug1.reshape(HEADS * NPAD, F)
    ast1 = as1.T.reshape(HEADS, NPAD)
    adt1 = ad1.T.reshape(HEADS, NPAD)
    cct1 = cc1.T.reshape(HEADS, NPAD)
    part1 = _sc_aggregate(HEADS, htab1, ast1, adt1, cct1, srca, dsta)

    b1_2d = b1.reshape(HEADS, HID)
    haug2, as2, ad2 = _l2_prep(part1, b1_2d, W2, a_src2, a_dst2)
    cc2 = _shift2_table(as2, ad2)

    ast2 = as2.reshape(1, NPAD)
    adt2 = ad2.reshape(1, NPAD)
    cct2 = cc2.reshape(1, NPAD)
    part2 = _sc_aggregate(1, haug2, ast2, adt2, cct2, srca, dsta)

    return _final(part2, b2.reshape(1, EMB))


# speedup vs baseline: 11.9250x; 11.9250x over previous
import functools

import jax
import jax.numpy as jnp
from jax import lax
from jax.experimental import pallas as pl
from jax.experimental.pallas import tpu as pltpu
from jax.experimental.pallas import tpu_sc as plsc

N = 10000
NPAD = 10240
E = 320000
D = 128
HID = 128
HEADS = 8
EMB = 128
F = 144
NTILES = 32
NBLK = 9
BCH = 18
CK = 64
EPAD = NTILES * NBLK * BCH * CK
ROWS_PER_TILE = NPAD // 16
ZROWS = 16


def _leaky(u):
    return jnp.maximum(u, 0.2 * u)


def _l1_prep_body(x_ref, w1_ref, asbd_ref, adbd_ref,
                  haug_ref, as_ref, ad_ref, am_ref):
    i = pl.program_id(0)
    h = jnp.dot(x_ref[...], w1_ref[...], preferred_element_type=jnp.float32)
    as_blk = jnp.dot(h, asbd_ref[...], preferred_element_type=jnp.float32)
    as_ref[...] = as_blk
    ad_ref[...] = jnp.dot(h, adbd_ref[...], preferred_element_type=jnp.float32)
    pad = jnp.concatenate(
        [jnp.ones((128, 1), jnp.float32), jnp.zeros((128, 15), jnp.float32)], axis=1)
    for hd in range(HEADS):
        haug_ref[hd, :, 0:128] = h[:, hd * 128:(hd + 1) * 128]
        haug_ref[hd, :, 128:144] = pad

    ridx = i * 128 + lax.broadcasted_iota(jnp.int32, (128, HEADS), 0)
    masked = jnp.where(ridx < N, as_blk, -3e38)
    bmax = jnp.max(masked, axis=0, keepdims=True)
    cur = jnp.broadcast_to(bmax, (16, HEADS))

    @pl.when(i == 0)
    def _():
        am_ref[...] = cur

    @pl.when(i > 0)
    def _():
        am_ref[...] = jnp.maximum(am_ref[...], cur)


def _l1_prep(xpad, W1, asbd, adbd):
    return pl.pallas_call(
        _l1_prep_body,
        grid=(NPAD // 128,),
        in_specs=[
            pl.BlockSpec((128, D), lambda i: (i, 0)),
            pl.BlockSpec((D, HEADS * HID), lambda i: (0, 0)),
            pl.BlockSpec((HEADS * HID, HEADS), lambda i: (0, 0)),
            pl.BlockSpec((HEADS * HID, HEADS), lambda i: (0, 0)),
        ],
        out_specs=[
            pl.BlockSpec((HEADS, 128, F), lambda i: (0, i, 0)),
            pl.BlockSpec((128, HEADS), lambda i: (i, 0)),
            pl.BlockSpec((128, HEADS), lambda i: (i, 0)),
            pl.BlockSpec((16, HEADS), lambda i: (0, 0)),
        ],
        out_shape=[
            jax.ShapeDtypeStruct((HEADS, NPAD, F), jnp.float32),
            jax.ShapeDtypeStruct((NPAD, HEADS), jnp.float32),
            jax.ShapeDtypeStruct((NPAD, HEADS), jnp.float32),
            jax.ShapeDtypeStruct((16, HEADS), jnp.float32),
        ],
    )(xpad, W1, asbd, adbd)


def _sc_body(nheads, htab, ast, adt, amx, srca, dsta, out_ref,
             sblk_v, dblk_v, gidx_v, w_v, rows_v, as_v, ad_v, am_v, zbuf_v,
             out_sp, sem):
    c = lax.axis_index("c")
    s = lax.axis_index("s")
    wid = c * 16 + s

    zero16 = jnp.zeros((16,), jnp.float32)

    def zb(i, _):
        for j in range(F // 16):
            zbuf_v[i, pl.ds(j * 16, 16)] = zero16
        return 0
    lax.fori_loop(0, ZROWS, zb, 0)

    def head_loop(hd, _):
        pltpu.sync_copy(ast.at[hd], as_v)
        pltpu.sync_copy(adt.at[hd], ad_v)
        pltpu.sync_copy(amx.at[hd], am_v)

        def zo(i, _):
            pltpu.sync_copy(zbuf_v,
                            out_sp.at[pl.ds(s * ROWS_PER_TILE + i * ZROWS, ZROWS)])
            return 0
        lax.fori_loop(0, ROWS_PER_TILE // ZROWS, zo, 0)
        plsc.subcore_barrier()

        base = hd * NPAD
        mvec = am_v[...]

        def blk_loop(blk, _):
            pltpu.sync_copy(srca.at[wid, blk], sblk_v)
            pltpu.sync_copy(dsta.at[wid, blk], dblk_v)

            def chunk_loop(ch, _):
                for j in range(CK // 16):
                    sl = pl.ds(j * 16, 16)
                    sv = sblk_v[ch, sl]
                    dv = dblk_v[ch, sl]
                    asg = plsc.load_gather(as_v, [sv])
                    adg = plsc.load_gather(ad_v, [dv])
                    u = asg + adg
                    cc = _leaky(adg + mvec)
                    w_v[sl] = jnp.exp(_leaky(u) - cc)
                    gidx_v[sl] = sv + base
                pltpu.async_copy(htab.at[gidx_v], rows_v, sem).wait()

                def scale(k, _):
                    wk = plsc.load_gather(w_v, [jnp.full((16,), k, jnp.int32)])
                    for j in range(F // 16):
                        fsl = pl.ds(j * 16, 16)
                        rows_v[k, fsl] = rows_v[k, fsl] * wk
                    return 0
                lax.fori_loop(0, CK, scale, 0)
                pltpu.sync_copy(rows_v, out_sp.at[dblk_v.at[ch]], add=True)
                return 0
            lax.fori_loop(0, BCH, chunk_loop, 0)
            return 0
        lax.fori_loop(0, NBLK, blk_loop, 0)
        plsc.subcore_barrier()

        def wo(i, _):
            r = s * ROWS_PER_TILE + i * 64
            pltpu.sync_copy(out_sp.at[pl.ds(r, 64)],
                            out_ref.at[c, hd, pl.ds(r, 64)])
            return 0
        lax.fori_loop(0, ROWS_PER_TILE // 64, wo, 0)
        plsc.subcore_barrier()
        return 0
    lax.fori_loop(0, nheads, head_loop, 0)


def _sc_aggregate(nheads, htab, ast, adt, amx, srca, dsta):
    mesh = plsc.VectorSubcoreMesh(core_axis_name="c", subcore_axis_name="s")
    kfn = pl.kernel(
        functools.partial(_sc_body, nheads),
        out_type=jax.ShapeDtypeStruct((2, nheads, NPAD, F), jnp.float32),
        mesh=mesh,
        scratch_types=[
            pltpu.VMEM((BCH, CK), jnp.int32),
            pltpu.VMEM((BCH, CK), jnp.int32),
            pltpu.VMEM((CK,), jnp.int32),
            pltpu.VMEM((CK,), jnp.float32),
            pltpu.VMEM((CK, F), jnp.float32),
            pltpu.VMEM((NPAD,), jnp.float32),
            pltpu.VMEM((NPAD,), jnp.float32),
            pltpu.VMEM((16,), jnp.float32),
            pltpu.VMEM((ZROWS, F), jnp.float32),
            pltpu.VMEM_SHARED((NPAD, F), jnp.float32),
            pltpu.SemaphoreType.DMA,
        ],
        compiler_params=pltpu.CompilerParams(
            needs_layout_passes=False, use_tc_tiling_on_sc=False),
    )
    return kfn(htab, ast, adt, amx, srca, dsta)


def _l2_prep_body(p_ref, b1_ref, w2_ref, as2v_ref, ad2v_ref,
                  haug_ref, as2_ref, ad2_ref, am_ref):
    i = pl.program_id(0)
    h2 = jnp.zeros((128, EMB), jnp.float32)
    for hd in range(HEADS):
        num = p_ref[0, hd, :, 0:128] + p_ref[1, hd, :, 0:128]
        den = p_ref[0, hd, :, 128:129] + p_ref[1, hd, :, 128:129]
        g = num / (den + 1e-16) + b1_ref[hd:hd + 1, :]
        e = jnp.where(g > 0, g, jnp.exp(g) - 1.0)
        h2 = h2 + jnp.dot(e, w2_ref[hd * 128:(hd + 1) * 128, :],
                          preferred_element_type=jnp.float32)
    pad = jnp.concatenate(
        [jnp.ones((128, 1), jnp.float32), jnp.zeros((128, 15), jnp.float32)], axis=1)
    haug_ref[:, 0:128] = h2
    haug_ref[:, 128:144] = pad
    as2_blk = jnp.sum(h2 * as2v_ref[0:1, :], axis=1, keepdims=True)
    as2_ref[0, 0, :] = as2_blk[:, 0]
    ad2_ref[0, 0, :] = jnp.sum(h2 * ad2v_ref[0:1, :], axis=1)

    ridx = i * 128 + lax.broadcasted_iota(jnp.int32, (128, 1), 0)
    masked = jnp.where(ridx < N, as2_blk, -3e38)
    cur = jnp.broadcast_to(jnp.max(masked, axis=0, keepdims=True), (16, 1))

    @pl.when(i == 0)
    def _():
        am_ref[...] = cur

    @pl.when(i > 0)
    def _():
        am_ref[...] = jnp.maximum(am_ref[...], cur)


def _l2_prep(part1, b1_2d, W2, a_src2, a_dst2):
    return pl.pallas_call(
        _l2_prep_body,
        grid=(NPAD // 128,),
        in_specs=[
            pl.BlockSpec((2, HEADS, 128, F), lambda i: (0, 0, i, 0)),
            pl.BlockSpec((HEADS, HID), lambda i: (0, 0)),
            pl.BlockSpec((HEADS * HID, EMB), lambda i: (0, 0)),
            pl.BlockSpec((1, EMB), lambda i: (0, 0)),
            pl.BlockSpec((1, EMB), lambda i: (0, 0)),
        ],
        out_specs=[
            pl.BlockSpec((128, F), lambda i: (i, 0)),
            pl.BlockSpec((1, 1, 128), lambda i: (i, 0, 0)),
            pl.BlockSpec((1, 1, 128), lambda i: (i, 0, 0)),
            pl.BlockSpec((16, 1), lambda i: (0, 0)),
        ],
        out_shape=[
            jax.ShapeDtypeStruct((NPAD, F), jnp.float32),
            jax.ShapeDtypeStruct((NPAD // 128, 1, 128), jnp.float32),
            jax.ShapeDtypeStruct((NPAD // 128, 1, 128), jnp.float32),
            jax.ShapeDtypeStruct((16, 1), jnp.float32),
        ],
    )(part1, b1_2d, W2, a_src2, a_dst2)


def _final_body(p_ref, b2_ref, out_ref):
    num = p_ref[0, :, 0:128] + p_ref[1, :, 0:128]
    den = p_ref[0, :, 128:129] + p_ref[1, :, 128:129]
    z = num / (den + 1e-16) + b2_ref[0:1, :]
    m = jnp.max(z, axis=1, keepdims=True)
    zs = z - m
    out_ref[...] = zs - jnp.log(jnp.sum(jnp.exp(zs), axis=1, keepdims=True))


def _final(part2, b2_2d):
    return pl.pallas_call(
        _final_body,
        grid=(79,),
        in_specs=[
            pl.BlockSpec((2, 128, F), lambda i: (0, i, 0)),
            pl.BlockSpec((1, EMB), lambda i: (0, 0)),
        ],
        out_specs=pl.BlockSpec((128, EMB), lambda i: (i, 0)),
        out_shape=jax.ShapeDtypeStruct((N, EMB), jnp.float32),
    )(part2, b2_2d)


def kernel(x, edge_index, W1, a_src1, a_dst1, b1, W2, a_src2, a_dst2, b2):
    ei = edge_index.astype(jnp.int32)
    loop = jnp.arange(N, dtype=jnp.int32)
    padn = EPAD - (E + N)
    src = jnp.concatenate([ei[0], loop, jnp.zeros((padn,), jnp.int32)])
    dst = jnp.concatenate([ei[1], loop, jnp.full((padn,), N + 10, jnp.int32)])
    srca = src.reshape(NTILES, NBLK, BCH, CK)
    dsta = dst.reshape(NTILES, NBLK, BCH, CK)

    xpad = jnp.pad(x, ((0, NPAD - N), (0, 0)))

    eye = jnp.eye(HEADS, dtype=jnp.float32)
    asbd = (eye[:, None, :] * a_src1[:, :, None]).reshape(HEADS * HID, HEADS)
    adbd = (eye[:, None, :] * a_dst1[:, :, None]).reshape(HEADS * HID, HEADS)

    haug1, as1, ad1, am1 = _l1_prep(xpad, W1, asbd, adbd)

    htab1 = haug1.reshape(HEADS * NPAD, F)
    ast1 = as1.T.reshape(HEADS, NPAD)
    adt1 = ad1.T.reshape(HEADS, NPAD)
    amx1 = am1.T.reshape(HEADS, 16)
    part1 = _sc_aggregate(HEADS, htab1, ast1, adt1, amx1, srca, dsta)

    b1_2d = b1.reshape(HEADS, HID)
    haug2, as2, ad2, am2 = _l2_prep(part1, b1_2d, W2, a_src2, a_dst2)

    ast2 = as2.reshape(1, NPAD)
    adt2 = ad2.reshape(1, NPAD)
    amx2 = am2.T.reshape(1, 16)
    part2 = _sc_aggregate(1, haug2, ast2, adt2, amx2, srca, dsta)

    return _final(part2.reshape(2, NPAD, F), b2.reshape(1, EMB))


# speedup vs baseline: 18.6520x; 1.5641x over previous
import functools

import jax
import jax.numpy as jnp
from jax import lax
from jax.experimental import pallas as pl
from jax.experimental.pallas import tpu as pltpu
from jax.experimental.pallas import tpu_sc as plsc

N = 10000
NPAD = 10240
E = 320000
D = 128
HID = 128
HEADS = 8
EMB = 128
F = 144
NTILES = 32
NBLK = 6
BCH = 27
CK = 64
BLKE = BCH * CK
EPAD = NTILES * NBLK * BLKE
ROWS_PER_TILE = NPAD // 16


def _leaky(u):
    return jnp.maximum(u, 0.2 * u)


def _l1_prep_body(x_ref, w1_ref, asbd_ref, adbd_ref,
                  haug_ref, as_ref, ad_ref, am_ref):
    i = pl.program_id(0)
    h = jnp.dot(x_ref[...], w1_ref[...], preferred_element_type=jnp.float32)
    as_blk = jnp.dot(h, asbd_ref[...], preferred_element_type=jnp.float32)
    as_ref[...] = as_blk
    ad_ref[...] = jnp.dot(h, adbd_ref[...], preferred_element_type=jnp.float32)
    pad = jnp.concatenate(
        [jnp.ones((128, 1), jnp.float32), jnp.zeros((128, 15), jnp.float32)], axis=1)
    for hd in range(HEADS):
        haug_ref[hd, :, 0:128] = h[:, hd * 128:(hd + 1) * 128]
        haug_ref[hd, :, 128:144] = pad

    ridx = i * 128 + lax.broadcasted_iota(jnp.int32, (128, HEADS), 0)
    masked = jnp.where(ridx < N, as_blk, -3e38)
    bmax = jnp.max(masked, axis=0, keepdims=True)
    cur = jnp.broadcast_to(bmax, (16, HEADS))

    @pl.when(i == 0)
    def _():
        am_ref[...] = cur

    @pl.when(i > 0)
    def _():
        am_ref[...] = jnp.maximum(am_ref[...], cur)


def _l1_prep(xpad, W1, asbd, adbd):
    return pl.pallas_call(
        _l1_prep_body,
        grid=(NPAD // 128,),
        in_specs=[
            pl.BlockSpec((128, D), lambda i: (i, 0)),
            pl.BlockSpec((D, HEADS * HID), lambda i: (0, 0)),
            pl.BlockSpec((HEADS * HID, HEADS), lambda i: (0, 0)),
            pl.BlockSpec((HEADS * HID, HEADS), lambda i: (0, 0)),
        ],
        out_specs=[
            pl.BlockSpec((HEADS, 128, F), lambda i: (0, i, 0)),
            pl.BlockSpec((128, HEADS), lambda i: (i, 0)),
            pl.BlockSpec((128, HEADS), lambda i: (i, 0)),
            pl.BlockSpec((16, HEADS), lambda i: (0, 0)),
        ],
        out_shape=[
            jax.ShapeDtypeStruct((HEADS, NPAD, F), jnp.float32),
            jax.ShapeDtypeStruct((NPAD, HEADS), jnp.float32),
            jax.ShapeDtypeStruct((NPAD, HEADS), jnp.float32),
            jax.ShapeDtypeStruct((16, HEADS), jnp.float32),
        ],
    )(xpad, W1, asbd, adbd)


def _sc_wk_body(nheads, ast, adt, amx, srcf, dstf, wout_ref, gout_ref,
                as_v, ad_v, am_v, sblk_v, dblk_v, wv, gv):
    c = lax.axis_index("c")
    s = lax.axis_index("s")
    wid = c * 16 + s

    def head_loop(hd, _):
        pltpu.sync_copy(ast.at[hd], as_v)
        pltpu.sync_copy(adt.at[hd], ad_v)
        pltpu.sync_copy(amx.at[hd], am_v)
        mvec = am_v[...]
        base = hd * NPAD

        def blk_loop(blk, _):
            pltpu.sync_copy(srcf.at[wid, blk], sblk_v)
            pltpu.sync_copy(dstf.at[wid, blk], dblk_v)

            def grp(g, _):
                sl = pl.ds(g * 16, 16)
                sv = sblk_v[sl]
                dv = dblk_v[sl]
                asg = plsc.load_gather(as_v, [sv])
                adg = plsc.load_gather(ad_v, [dv])
                u = asg + adg
                cc = _leaky(adg + mvec)
                wv[sl] = jnp.exp(_leaky(u) - cc)
                gv[sl] = sv + base
                return 0
            lax.fori_loop(0, BLKE // 16, grp, 0)
            pltpu.sync_copy(wv, wout_ref.at[hd, wid, blk])
            pltpu.sync_copy(gv, gout_ref.at[hd, wid, blk])
            return 0
        lax.fori_loop(0, NBLK, blk_loop, 0)
        return 0
    lax.fori_loop(0, nheads, head_loop, 0)


def _sc_wk(nheads, ast, adt, amx, srcf, dstf):
    mesh = plsc.VectorSubcoreMesh(core_axis_name="c", subcore_axis_name="s")
    kfn = pl.kernel(
        functools.partial(_sc_wk_body, nheads),
        out_type=[
            jax.ShapeDtypeStruct((nheads, NTILES, NBLK, BLKE), jnp.float32),
            jax.ShapeDtypeStruct((nheads, NTILES, NBLK, BLKE), jnp.int32),
        ],
        mesh=mesh,
        scratch_types=[
            pltpu.VMEM((NPAD,), jnp.float32),
            pltpu.VMEM((NPAD,), jnp.float32),
            pltpu.VMEM((16,), jnp.float32),
            pltpu.VMEM((BLKE,), jnp.int32),
            pltpu.VMEM((BLKE,), jnp.int32),
            pltpu.VMEM((BLKE,), jnp.float32),
            pltpu.VMEM((BLKE,), jnp.int32),
        ],
        compiler_params=pltpu.CompilerParams(
            needs_layout_passes=False, use_tc_tiling_on_sc=False),
    )
    return kfn(ast, adt, amx, srcf, dstf)


def _sc_agg_body(nheads, htab, g5, w4, d5, out_ref,
                 rows0, rows1, rows2, dblk_v, gblk_v, wblk_v,
                 gs0, gs1, gs2, ss0, ss1, ss2,
                 out_sp):
    c = lax.axis_index("c")
    s = lax.axis_index("s")
    wid = c * 16 + s
    rows = (rows0, rows1, rows2)
    gsem = (gs0, gs1, gs2)
    ssem = (ss0, ss1, ss2)
    zero16 = jnp.zeros((16,), jnp.float32)

    def head_loop(hd, _):
        def zr(i, _):
            for j in range(F // 16):
                rows2[i, pl.ds(j * 16, 16)] = zero16
            return 0
        lax.fori_loop(0, CK, zr, 0)

        def zo(i, _):
            pltpu.sync_copy(rows2,
                            out_sp.at[pl.ds(s * ROWS_PER_TILE + i * CK, CK)])
            return 0
        lax.fori_loop(0, ROWS_PER_TILE // CK, zo, 0)
        plsc.subcore_barrier()

        def blk_loop(blk, _):
            pltpu.sync_copy(d5.at[wid, blk], dblk_v)
            pltpu.sync_copy(g5.at[hd, wid, blk], gblk_v)
            pltpu.sync_copy(w4.at[hd, wid, blk], wblk_v)
            pltpu.async_copy(htab.at[gblk_v.at[0]], rows0, gs0)
            pltpu.async_copy(htab.at[gblk_v.at[1]], rows1, gs1)

            def step(ch, b, wait_prev, prefetch):
                rb = rows[b]
                pltpu.make_async_copy(htab.at[gblk_v.at[ch]], rb,
                                      gsem[b]).wait()

                def scale(k, _):
                    wk = plsc.load_gather(
                        wblk_v, [jnp.full((16,), ch * CK, jnp.int32)
                                 + jnp.full((16,), k, jnp.int32)])
                    for j in range(F // 16):
                        fsl = pl.ds(j * 16, 16)
                        rb[k, fsl] = rb[k, fsl] * wk
                    return 0
                lax.fori_loop(0, CK, scale, 0)

                bn = (b + 2) % 3

                @pl.when(prefetch)
                def _():
                    @pl.when(wait_prev)
                    def _():
                        pltpu.make_async_copy(
                            rows[bn], out_sp.at[dblk_v.at[ch - 1]],
                            ssem[bn]).wait()
                    pltpu.async_copy(htab.at[gblk_v.at[ch + 2]], rows[bn],
                                     gsem[bn])

                pltpu.async_copy(rb, out_sp.at[dblk_v.at[ch]], ssem[b],
                                 add=True)

            true_ = jnp.bool_(True)

            def triple(t, _):
                step(3 * t, 0, t >= 1, true_)
                step(3 * t + 1, 1, true_, t < 8)
                step(3 * t + 2, 2, true_, t < 8)
                return 0
            lax.fori_loop(0, BCH // 3, triple, 0)
            for b, chl in ((0, BCH - 3), (1, BCH - 2), (2, BCH - 1)):
                pltpu.make_async_copy(rows[b], out_sp.at[dblk_v.at[chl]],
                                      ssem[b]).wait()
            return 0
        lax.fori_loop(0, NBLK, blk_loop, 0)
        plsc.subcore_barrier()

        def wo(i, _):
            r = s * ROWS_PER_TILE + i * 64
            pltpu.sync_copy(out_sp.at[pl.ds(r, 64)],
                            out_ref.at[c, hd, pl.ds(r, 64)])
            return 0
        lax.fori_loop(0, ROWS_PER_TILE // 64, wo, 0)
        plsc.subcore_barrier()
        return 0
    lax.fori_loop(0, nheads, head_loop, 0)


def _sc_aggregate(nheads, htab, ast, adt, amx, srcf, dstf, d5):
    w4, g4 = _sc_wk(nheads, ast, adt, amx, srcf, dstf)
    g5 = g4.reshape(nheads, NTILES, NBLK, BCH, CK)
    mesh = plsc.VectorSubcoreMesh(core_axis_name="c", subcore_axis_name="s")
    kfn = pl.kernel(
        functools.partial(_sc_agg_body, nheads),
        out_type=jax.ShapeDtypeStruct((2, nheads, NPAD, F), jnp.float32),
        mesh=mesh,
        scratch_types=[
            pltpu.VMEM((CK, F), jnp.float32),
            pltpu.VMEM((CK, F), jnp.float32),
            pltpu.VMEM((CK, F), jnp.float32),
            pltpu.VMEM((BCH, CK), jnp.int32),
            pltpu.VMEM((BCH, CK), jnp.int32),
            pltpu.VMEM((BLKE,), jnp.float32),
            pltpu.SemaphoreType.DMA,
            pltpu.SemaphoreType.DMA,
            pltpu.SemaphoreType.DMA,
            pltpu.SemaphoreType.DMA,
            pltpu.SemaphoreType.DMA,
            pltpu.SemaphoreType.DMA,
            pltpu.VMEM_SHARED((NPAD, F), jnp.float32),
        ],
        compiler_params=pltpu.CompilerParams(
            needs_layout_passes=False, use_tc_tiling_on_sc=False),
    )
    return kfn(htab, g5, w4, d5)


def _l2_prep_body(p_ref, b1_ref, w2_ref, as2v_ref, ad2v_ref,
                  haug_ref, as2_ref, ad2_ref, am_ref):
    i = pl.program_id(0)
    h2 = jnp.zeros((128, EMB), jnp.float32)
    for hd in range(HEADS):
        num = p_ref[0, hd, :, 0:128] + p_ref[1, hd, :, 0:128]
        den = p_ref[0, hd, :, 128:129] + p_ref[1, hd, :, 128:129]
        g = num / (den + 1e-16) + b1_ref[hd:hd + 1, :]
        e = jnp.where(g > 0, g, jnp.exp(g) - 1.0)
        h2 = h2 + jnp.dot(e, w2_ref[hd * 128:(hd + 1) * 128, :],
                          preferred_element_type=jnp.float32)
    pad = jnp.concatenate(
        [jnp.ones((128, 1), jnp.float32), jnp.zeros((128, 15), jnp.float32)], axis=1)
    haug_ref[:, 0:128] = h2
    haug_ref[:, 128:144] = pad
    as2_blk = jnp.sum(h2 * as2v_ref[0:1, :], axis=1, keepdims=True)
    as2_ref[0, 0, :] = as2_blk[:, 0]
    ad2_ref[0, 0, :] = jnp.sum(h2 * ad2v_ref[0:1, :], axis=1)

    ridx = i * 128 + lax.broadcasted_iota(jnp.int32, (128, 1), 0)
    masked = jnp.where(ridx < N, as2_blk, -3e38)
    cur = jnp.broadcast_to(jnp.max(masked, axis=0, keepdims=True), (16, 1))

    @pl.when(i == 0)
    def _():
        am_ref[...] = cur

    @pl.when(i > 0)
    def _():
        am_ref[...] = jnp.maximum(am_ref[...], cur)


def _l2_prep(part1, b1_2d, W2, a_src2, a_dst2):
    return pl.pallas_call(
        _l2_prep_body,
        grid=(NPAD // 128,),
        in_specs=[
            pl.BlockSpec((2, HEADS, 128, F), lambda i: (0, 0, i, 0)),
            pl.BlockSpec((HEADS, HID), lambda i: (0, 0)),
            pl.BlockSpec((HEADS * HID, EMB), lambda i: (0, 0)),
            pl.BlockSpec((1, EMB), lambda i: (0, 0)),
            pl.BlockSpec((1, EMB), lambda i: (0, 0)),
        ],
        out_specs=[
            pl.BlockSpec((128, F), lambda i: (i, 0)),
            pl.BlockSpec((1, 1, 128), lambda i: (i, 0, 0)),
            pl.BlockSpec((1, 1, 128), lambda i: (i, 0, 0)),
            pl.BlockSpec((16, 1), lambda i: (0, 0)),
        ],
        out_shape=[
            jax.ShapeDtypeStruct((NPAD, F), jnp.float32),
            jax.ShapeDtypeStruct((NPAD // 128, 1, 128), jnp.float32),
            jax.ShapeDtypeStruct((NPAD // 128, 1, 128), jnp.float32),
            jax.ShapeDtypeStruct((16, 1), jnp.float32),
        ],
    )(part1, b1_2d, W2, a_src2, a_dst2)


def _final_body(p_ref, b2_ref, out_ref):
    num = p_ref[0, :, 0:128] + p_ref[1, :, 0:128]
    den = p_ref[0, :, 128:129] + p_ref[1, :, 128:129]
    z = num / (den + 1e-16) + b2_ref[0:1, :]
    m = jnp.max(z, axis=1, keepdims=True)
    zs = z - m
    out_ref[...] = zs - jnp.log(jnp.sum(jnp.exp(zs), axis=1, keepdims=True))


def _final(part2, b2_2d):
    return pl.pallas_call(
        _final_body,
        grid=(79,),
        in_specs=[
            pl.BlockSpec((2, 128, F), lambda i: (0, i, 0)),
            pl.BlockSpec((1, EMB), lambda i: (0, 0)),
        ],
        out_specs=pl.BlockSpec((128, EMB), lambda i: (i, 0)),
        out_shape=jax.ShapeDtypeStruct((N, EMB), jnp.float32),
    )(part2, b2_2d)


def kernel(x, edge_index, W1, a_src1, a_dst1, b1, W2, a_src2, a_dst2, b2):
    ei = edge_index.astype(jnp.int32)
    loop = jnp.arange(N, dtype=jnp.int32)
    padn = EPAD - (E + N)
    src = jnp.concatenate([ei[0], loop, jnp.zeros((padn,), jnp.int32)])
    dst = jnp.concatenate([ei[1], loop, jnp.full((padn,), N + 10, jnp.int32)])
    srcf = src.reshape(NTILES, NBLK, BLKE)
    dstf = dst.reshape(NTILES, NBLK, BLKE)
    d5 = dst.reshape(NTILES, NBLK, BCH, CK)

    xpad = jnp.pad(x, ((0, NPAD - N), (0, 0)))

    eye = jnp.eye(HEADS, dtype=jnp.float32)
    asbd = (eye[:, None, :] * a_src1[:, :, None]).reshape(HEADS * HID, HEADS)
    adbd = (eye[:, None, :] * a_dst1[:, :, None]).reshape(HEADS * HID, HEADS)

    haug1, as1, ad1, am1 = _l1_prep(xpad, W1, asbd, adbd)

    htab1 = haug1.reshape(HEADS * NPAD, F)
    ast1 = as1.T.reshape(HEADS, NPAD)
    adt1 = ad1.T.reshape(HEADS, NPAD)
    amx1 = am1.T.reshape(HEADS, 16)
    part1 = _sc_aggregate(HEADS, htab1, ast1, adt1, amx1, srcf, dstf, d5)

    b1_2d = b1.reshape(HEADS, HID)
    haug2, as2, ad2, am2 = _l2_prep(part1, b1_2d, W2, a_src2, a_dst2)

    ast2 = as2.reshape(1, NPAD)
    adt2 = ad2.reshape(1, NPAD)
    amx2 = am2.T.reshape(1, 16)
    part2 = _sc_aggregate(1, haug2, ast2, adt2, amx2, srcf, dstf, d5)

    return _final(part2.reshape(2, NPAD, F), b2.reshape(1, EMB))
